# baseline (speedup 1.0000x reference)
import jax
import jax.numpy as jnp
from jax import lax
from jax.experimental import pallas as pl
from jax.experimental.pallas import tpu as pltpu
from jax.experimental.pallas import tpu_sc as plsc

_F32 = jnp.float32
_NC = 2
_NS = 16
_NW = _NC * _NS
_K = 128


def _sc_mesh():
    return plsc.VectorSubcoreMesh(core_axis_name="c", subcore_axis_name="s")


def _zero_fill_1d(ref, n):
    def body(i, carry):
        ref[pl.ds(i * 16, 16)] = jnp.zeros((16,), _F32)
        return carry
    lax.fori_loop(0, n // 16, body, 0)


def _deg_kernel(npad, nbt, rpt):
    nb = nbt // _NW
    dep = 8

    def body(er_hbm, deg_out, deg_sh, idx_v, ones_v, lin_v, ssem):
        c = lax.axis_index("c")
        s = lax.axis_index("s")
        wid = s * _NC + c
        _zero_fill_1d(lin_v, rpt)
        def wo(i, carry):
            ones_v[pl.ds(i * 16, 16)] = jnp.ones((16,), _F32)
            return carry
        lax.fori_loop(0, _K // 16, wo, 0)
        pltpu.sync_copy(lin_v, deg_sh.at[pl.ds(s * rpt, rpt)])
        plsc.subcore_barrier()
        pltpu.sync_copy(er_hbm.at[1, pl.ds(wid * nb, nb)], idx_v)
        def group(i, carry):
            descs = [
                pltpu.async_copy(ones_v, deg_sh.at[idx_v.at[i * dep + j]],
                                 ssem, add=True)
                for j in range(dep)
            ]
            for dsc in descs:
                dsc.wait()
            return carry
        lax.fori_loop(0, nb // dep, group, 0)
        plsc.subcore_barrier()
        pltpu.sync_copy(deg_sh.at[pl.ds(s * rpt, rpt)], lin_v)
        pltpu.sync_copy(lin_v, deg_out.at[c, pl.ds(s * rpt, rpt)])

    return pl.kernel(
        body,
        out_type=jax.ShapeDtypeStruct((_NC, npad), _F32),
        mesh=_sc_mesh(),
        scratch_types=[
            pltpu.VMEM_SHARED((npad,), _F32),
            pltpu.VMEM((nb, _K), jnp.int32),
            pltpu.VMEM((_K,), _F32),
            pltpu.VMEM((rpt,), _F32),
            pltpu.SemaphoreType.DMA,
        ],
    )


def _agg_kernel(npad, d, nbt, rpt):
    nb = nbt // _NW
    hb = nb // 2
    nchunk = rpt // 80

    def body(er_hbm, h2_hbm, s0_out, s1_out, s_sh, src_v, dst_v,
             rows0, rows1, gsem, ssem):
        c = lax.axis_index("c")
        s = lax.axis_index("s")
        wid = s * _NC + c
        def zr(i, carry):
            rr = i // (d // 16)
            col = i % (d // 16)
            rows0[rr, pl.ds(col * 16, 16)] = jnp.zeros((16,), _F32)
            return carry
        lax.fori_loop(0, 80 * (d // 16), zr, 0)
        def zcp(j, carry):
            pltpu.sync_copy(rows0.at[pl.ds(0, 80)],
                            s_sh.at[pl.ds(s * rpt + j * 80, 80)])
            return carry
        lax.fori_loop(0, nchunk, zcp, 0)
        plsc.subcore_barrier()

        def gather(b, rows):
            return pltpu.async_copy(h2_hbm.at[src_v.at[b]], rows, gsem)

        def scatter(b, rows):
            return pltpu.async_copy(rows, s_sh.at[dst_v.at[b]], ssem,
                                    add=True)

        for half in range(2):
            base = wid * nb + half * hb
            pltpu.sync_copy(er_hbm.at[0, pl.ds(base, hb)], src_v)
            pltpu.sync_copy(er_hbm.at[1, pl.ds(base, hb)], dst_v)
            d0 = gather(0, rows0)
            d1 = gather(1, rows1)
            d0.wait()
            s0 = scatter(0, rows0)
            d1.wait()
            s1 = scatter(1, rows1)
            for k in range(2, hb, 2):
                s0.wait()
                d0 = gather(k, rows0)
                s1.wait()
                d1 = gather(k + 1, rows1)
                d0.wait()
                s0 = scatter(k, rows0)
                d1.wait()
                s1 = scatter(k + 1, rows1)
            s0.wait()
            s1.wait()
        plsc.subcore_barrier()

        def dump(out_ref):
            def dj(j, carry):
                off = s * rpt + j * 80
                pltpu.sync_copy(s_sh.at[pl.ds(off, 80)],
                                rows0.at[pl.ds(0, 80)])
                pltpu.sync_copy(rows0.at[pl.ds(0, 80)],
                                out_ref.at[pl.ds(off, 80)])
                return carry
            lax.fori_loop(0, nchunk, dj, 0)
        @pl.when(c == 0)
        def _():
            dump(s0_out)
        @pl.when(c == 1)
        def _():
            dump(s1_out)

    return pl.kernel(
        body,
        out_type=[jax.ShapeDtypeStruct((npad, d), _F32),
                  jax.ShapeDtypeStruct((npad, d), _F32)],
        mesh=_sc_mesh(),
        scratch_types=[
            pltpu.VMEM_SHARED((npad, d), _F32),
            pltpu.VMEM((hb, _K), jnp.int32),
            pltpu.VMEM((hb, _K), jnp.int32),
            pltpu.VMEM((_K, d), _F32),
            pltpu.VMEM((_K, d), _F32),
            pltpu.SemaphoreType.DMA,
            pltpu.SemaphoreType.DMA,
        ],
    )


def _deg_col(deg_ref):
    ones = jnp.ones((2, 1), _F32)
    return lax.dot_general(deg_ref[...], ones, (((0,), (0,)), ((), ())),
                           preferred_element_type=_F32) + 1.0


def _tca_body(n, r, dout, rank):
    def body(x_ref, wg_ref, wa_ref, ba_ref, deg_ref, h2_ref, tmp_ref,
             vtz_ref, cucv_ref):
        i = pl.program_id(0)
        xb = x_ref[...]
        dinv = lax.rsqrt(jnp.maximum(_deg_col(deg_ref), 1e-12))
        h2_ref[...] = jnp.dot(xb, wg_ref[...],
                              preferred_element_type=_F32) * dinv
        tmpb = jnp.maximum(
            jnp.dot(xb, wa_ref[...], preferred_element_type=_F32)
            + ba_ref[...], 0.0)
        tmp_ref[...] = jnp.concatenate(
            [tmpb[:, 0:rank], tmpb[:, 3 * rank:]], axis=1)
        rows = lax.broadcasted_iota(jnp.int32, (r, 1), 0) + i * r
        mask = rows < n
        um = jnp.where(mask, tmpb[:, 0:rank], 0.0)
        vm = jnp.where(mask, tmpb[:, rank:2 * rank], 0.0)
        zb = tmpb[:, 2 * rank:3 * rank]
        vtz_b = lax.dot_general(vm, zb, (((0,), (0,)), ((), ())),
                                preferred_element_type=_F32)
        cucv_b = jnp.concatenate(
            [jnp.sum(um, axis=0, keepdims=True),
             jnp.sum(vm, axis=0, keepdims=True)], axis=0)
        @pl.when(i == 0)
        def _init():
            vtz_ref[...] = vtz_b
            cucv_ref[...] = cucv_b
        @pl.when(i > 0)
        def _acc():
            vtz_ref[...] += vtz_b
            cucv_ref[...] += cucv_b
    return body


def _tcc_body(n, dout, rank):
    def body(s0_ref, s1_ref, h2_ref, tmp_ref, deg_ref, vtz_ref, cucv_ref,
             wr_ref, bg_ref, br_ref, out_ref):
        dinv = lax.rsqrt(jnp.maximum(_deg_col(deg_ref), 1e-12))
        agg = dinv * (s0_ref[...] + s1_ref[...] + h2_ref[...]) + bg_ref[...]
        x_local = jnp.maximum(agg, 0.0)
        cu = cucv_ref[0:1, :]
        cv = cucv_ref[1:2, :]
        dn = float(n) / jnp.sum(cu * cv)
        wr1 = wr_ref[0:dout, :]
        wr2 = wr_ref[dout:dout + rank, :]
        wr3 = wr_ref[dout + rank:, :]
        m = jnp.dot(vtz_ref[...], wr2, preferred_element_type=_F32) * dn
        tmpb = tmp_ref[...]
        u = tmpb[:, 0:rank]
        t = tmpb[:, rank:]
        out_ref[...] = (
            jnp.dot(x_local, wr1, preferred_element_type=_F32)
            + jnp.dot(u, m, preferred_element_type=_F32)
            + jnp.dot(t, wr3, preferred_element_type=_F32)
            + br_ref[...])
    return body


def kernel(x, edge_index, W_gcn, b_gcn, W_att, b_att, W_red, b_red):
    n, d_in = x.shape
    e = edge_index.shape[1]
    dout = W_gcn.shape[1]
    fr = W_att.shape[1]
    rank = fr // 4
    npad = ((n + _NS * 128 - 1) // (_NS * 128)) * (_NS * 128)
    rpt = npad // _NS
    r = npad // 2
    g = npad // r

    quant = _K * _NW * 8
    epad = ((e + quant - 1) // quant) * quant
    nbt = epad // _K
    if epad != e:
        pe = epad - e
        ar = jnp.arange(pe, dtype=jnp.int32)
        pad = jnp.stack([ar % n, n + ar % (npad - n)])
        er = jnp.concatenate([edge_index, pad], axis=1)
    else:
        er = edge_index
    er = er.reshape(2, nbt, _K)

    deg_parts = _deg_kernel(npad, nbt, rpt)(er)

    h2, tmp, vtz, cucv = pl.pallas_call(
        _tca_body(n, r, dout, rank),
        grid=(g,),
        in_specs=[
            pl.BlockSpec((r, d_in), lambda i: (i, 0)),
            pl.BlockSpec((d_in, dout), lambda i: (0, 0)),
            pl.BlockSpec((d_in, fr), lambda i: (0, 0)),
            pl.BlockSpec((fr,), lambda i: (0,)),
            pl.BlockSpec((_NC, r), lambda i: (0, i)),
        ],
        out_specs=[
            pl.BlockSpec((r, dout), lambda i: (i, 0)),
            pl.BlockSpec((r, 2 * rank), lambda i: (i, 0)),
            pl.BlockSpec((rank, rank), lambda i: (0, 0)),
            pl.BlockSpec((2, rank), lambda i: (0, 0)),
        ],
        out_shape=[
            jax.ShapeDtypeStruct((npad, dout), _F32),
            jax.ShapeDtypeStruct((npad, 2 * rank), _F32),
            jax.ShapeDtypeStruct((rank, rank), _F32),
            jax.ShapeDtypeStruct((2, rank), _F32),
        ],
    )(x, W_gcn, W_att, b_att, deg_parts)

    s0, s1 = _agg_kernel(npad, dout, nbt, rpt)(er, h2)

    out = pl.pallas_call(
        _tcc_body(n, dout, rank),
        grid=(g,),
        in_specs=[
            pl.BlockSpec((r, dout), lambda i: (i, 0)),
            pl.BlockSpec((r, dout), lambda i: (i, 0)),
            pl.BlockSpec((r, dout), lambda i: (i, 0)),
            pl.BlockSpec((r, 2 * rank), lambda i: (i, 0)),
            pl.BlockSpec((_NC, r), lambda i: (0, i)),
            pl.BlockSpec((rank, rank), lambda i: (0, 0)),
            pl.BlockSpec((2, rank), lambda i: (0, 0)),
            pl.BlockSpec((dout + 2 * rank, dout), lambda i: (0, 0)),
            pl.BlockSpec((dout,), lambda i: (0,)),
            pl.BlockSpec((dout,), lambda i: (0,)),
        ],
        out_specs=pl.BlockSpec((r, dout), lambda i: (i, 0)),
        out_shape=jax.ShapeDtypeStruct((n, dout), _F32),
    )(s0, s1, h2, tmp, deg_parts, vtz, cucv, W_red, b_gcn, b_red)

    return out


# speedup vs baseline: 1.0012x; 1.0012x over previous
import jax
import jax.numpy as jnp
from jax import lax
from jax.experimental import pallas as pl
from jax.experimental.pallas import tpu as pltpu
from jax.experimental.pallas import tpu_sc as plsc

_F32 = jnp.float32
_NC = 2
_NS = 16
_NW = _NC * _NS
_K = 128


def _sc_mesh():
    return plsc.VectorSubcoreMesh(core_axis_name="c", subcore_axis_name="s")


def _zero_fill_1d(ref, n):
    def body(i, carry):
        ref[pl.ds(i * 16, 16)] = jnp.zeros((16,), _F32)
        return carry
    lax.fori_loop(0, n // 16, body, 0)


def _deg_kernel(npad, nbt, rpt):
    nb = nbt // _NW
    dep = 8

    def body(er_hbm, deg_out, deg_sh, idx_v, ones_v, lin_v, ssem):
        c = lax.axis_index("c")
        s = lax.axis_index("s")
        wid = s * _NC + c
        _zero_fill_1d(lin_v, rpt)
        def wo(i, carry):
            ones_v[pl.ds(i * 16, 16)] = jnp.ones((16,), _F32)
            return carry
        lax.fori_loop(0, _K // 16, wo, 0)
        pltpu.sync_copy(lin_v, deg_sh.at[pl.ds(s * rpt, rpt)])
        plsc.subcore_barrier()
        pltpu.sync_copy(er_hbm.at[1, pl.ds(wid * nb, nb)], idx_v)
        def group(i, carry):
            descs = [
                pltpu.async_copy(ones_v, deg_sh.at[idx_v.at[i * dep + j]],
                                 ssem, add=True)
                for j in range(dep)
            ]
            for dsc in descs:
                dsc.wait()
            return carry
        lax.fori_loop(0, nb // dep, group, 0)
        plsc.subcore_barrier()
        pltpu.sync_copy(deg_sh.at[pl.ds(s * rpt, rpt)], lin_v)
        pltpu.sync_copy(lin_v, deg_out.at[c, pl.ds(s * rpt, rpt)])

    return pl.kernel(
        body,
        out_type=jax.ShapeDtypeStruct((_NC, npad), _F32),
        mesh=_sc_mesh(),
        scratch_types=[
            pltpu.VMEM_SHARED((npad,), _F32),
            pltpu.VMEM((nb, _K), jnp.int32),
            pltpu.VMEM((_K,), _F32),
            pltpu.VMEM((rpt,), _F32),
            pltpu.SemaphoreType.DMA,
        ],
    )


def _agg_kernel(npad, d, nbt, rpt):
    nb = nbt // _NW
    hb = nb // 2
    nchunk = rpt // 80

    def body(er_hbm, h2_hbm, s0_out, s1_out, s_sh, src_v, dst_v,
             rows0, rows1, gsem, ssem):
        c = lax.axis_index("c")
        s = lax.axis_index("s")
        wid = s * _NC + c
        def zr(i, carry):
            rr = i // (d // 16)
            col = i % (d // 16)
            rows0[rr, pl.ds(col * 16, 16)] = jnp.zeros((16,), _F32)
            return carry
        lax.fori_loop(0, 80 * (d // 16), zr, 0)
        def zcp(j, carry):
            pltpu.sync_copy(rows0.at[pl.ds(0, 80)],
                            s_sh.at[pl.ds(s * rpt + j * 80, 80)])
            return carry
        lax.fori_loop(0, nchunk, zcp, 0)
        plsc.subcore_barrier()

        def gather(b, rows):
            return pltpu.async_copy(h2_hbm.at[src_v.at[b]], rows, gsem)

        def scatter(b, rows):
            return pltpu.async_copy(rows, s_sh.at[dst_v.at[b]], ssem,
                                    add=True)

        for half in range(2):
            base = wid * nb + half * hb
            pltpu.sync_copy(er_hbm.at[0, pl.ds(base, hb)], src_v)
            pltpu.sync_copy(er_hbm.at[1, pl.ds(base, hb)], dst_v)
            d0 = gather(0, rows0)
            d1 = gather(1, rows1)
            d0.wait()
            s0 = scatter(0, rows0)
            d1.wait()
            s1 = scatter(1, rows1)
            for k in range(2, hb, 2):
                s0.wait()
                d0 = gather(k, rows0)
                s1.wait()
                d1 = gather(k + 1, rows1)
                d0.wait()
                s0 = scatter(k, rows0)
                d1.wait()
                s1 = scatter(k + 1, rows1)
            s0.wait()
            s1.wait()
        plsc.subcore_barrier()

        def dump(out_ref):
            def dj(j, carry):
                off = s * rpt + j * 80
                pltpu.sync_copy(s_sh.at[pl.ds(off, 80)],
                                rows0.at[pl.ds(0, 80)])
                pltpu.sync_copy(rows0.at[pl.ds(0, 80)],
                                out_ref.at[pl.ds(off, 80)])
                return carry
            lax.fori_loop(0, nchunk, dj, 0)
        @pl.when(c == 0)
        def _():
            dump(s0_out)
        @pl.when(c == 1)
        def _():
            dump(s1_out)

    return pl.kernel(
        body,
        out_type=[jax.ShapeDtypeStruct((npad, d), _F32),
                  jax.ShapeDtypeStruct((npad, d), _F32)],
        mesh=_sc_mesh(),
        scratch_types=[
            pltpu.VMEM_SHARED((npad, d), _F32),
            pltpu.VMEM((hb, _K), jnp.int32),
            pltpu.VMEM((hb, _K), jnp.int32),
            pltpu.VMEM((_K, d), _F32),
            pltpu.VMEM((_K, d), _F32),
            pltpu.SemaphoreType.DMA,
            pltpu.SemaphoreType.DMA,
        ],
    )


def _deg_col(deg_ref):
    ones = jnp.ones((2, 1), _F32)
    return lax.dot_general(deg_ref[...], ones, (((0,), (0,)), ((), ())),
                           preferred_element_type=_F32) + 1.0


def _tca_body(n, r, dout, rank):
    def body(x_ref, wg_ref, wa_ref, ba_ref, deg_ref, h2_ref, tmp_ref,
             vtz_ref, cucv_ref):
        i = pl.program_id(0)
        xb = x_ref[...]
        dinv = lax.rsqrt(jnp.maximum(_deg_col(deg_ref), 1e-12))
        h2_ref[...] = jnp.dot(xb, wg_ref[...],
                              preferred_element_type=_F32) * dinv
        tmpb = jnp.maximum(
            jnp.dot(xb, wa_ref[...], preferred_element_type=_F32)
            + ba_ref[...], 0.0)
        tmp_ref[...] = jnp.concatenate(
            [tmpb[:, 0:rank], tmpb[:, 3 * rank:]], axis=1)
        rows = lax.broadcasted_iota(jnp.int32, (r, 1), 0) + i * r
        mask = rows < n
        um = jnp.where(mask, tmpb[:, 0:rank], 0.0)
        vm = jnp.where(mask, tmpb[:, rank:2 * rank], 0.0)
        zb = tmpb[:, 2 * rank:3 * rank]
        vtz_b = lax.dot_general(vm, zb, (((0,), (0,)), ((), ())),
                                preferred_element_type=_F32)
        cucv_b = jnp.concatenate(
            [jnp.sum(um, axis=0, keepdims=True),
             jnp.sum(vm, axis=0, keepdims=True)], axis=0)
        @pl.when(i == 0)
        def _init():
            vtz_ref[...] = vtz_b
            cucv_ref[...] = cucv_b
        @pl.when(i > 0)
        def _acc():
            vtz_ref[...] += vtz_b
            cucv_ref[...] += cucv_b
    return body


def _tcc_body(n, dout, rank):
    def body(s0_ref, s1_ref, h2_ref, tmp_ref, deg_ref, vtz_ref, cucv_ref,
             wr_ref, bg_ref, br_ref, out_ref):
        dinv = lax.rsqrt(jnp.maximum(_deg_col(deg_ref), 1e-12))
        agg = dinv * (s0_ref[...] + s1_ref[...] + h2_ref[...]) + bg_ref[...]
        x_local = jnp.maximum(agg, 0.0)
        cu = cucv_ref[0:1, :]
        cv = cucv_ref[1:2, :]
        dn = float(n) / jnp.sum(cu * cv)
        wr1 = wr_ref[0:dout, :]
        wr2 = wr_ref[dout:dout + rank, :]
        wr3 = wr_ref[dout + rank:, :]
        m = jnp.dot(vtz_ref[...], wr2, preferred_element_type=_F32) * dn
        tmpb = tmp_ref[...]
        u = tmpb[:, 0:rank]
        t = tmpb[:, rank:]
        out_ref[...] = (
            jnp.dot(x_local, wr1, preferred_element_type=_F32)
            + jnp.dot(u, m, preferred_element_type=_F32)
            + jnp.dot(t, wr3, preferred_element_type=_F32)
            + br_ref[...])
    return body


def kernel(x, edge_index, W_gcn, b_gcn, W_att, b_att, W_red, b_red):
    n, d_in = x.shape
    e = edge_index.shape[1]
    dout = W_gcn.shape[1]
    fr = W_att.shape[1]
    rank = fr // 4
    npad = ((n + _NS * 128 - 1) // (_NS * 128)) * (_NS * 128)
    rpt = npad // _NS
    r = npad // 4
    g = npad // r

    quant = _K * _NW * 8
    epad = ((e + quant - 1) // quant) * quant
    nbt = epad // _K
    if epad != e:
        pe = epad - e
        ar = jnp.arange(pe, dtype=jnp.int32)
        pad = jnp.stack([ar % n, n + ar % (npad - n)])
        er = jnp.concatenate([edge_index, pad], axis=1)
    else:
        er = edge_index
    er = er.reshape(2, nbt, _K)

    deg_parts = _deg_kernel(npad, nbt, rpt)(er)

    h2, tmp, vtz, cucv = pl.pallas_call(
        _tca_body(n, r, dout, rank),
        grid=(g,),
        in_specs=[
            pl.BlockSpec((r, d_in), lambda i: (i, 0)),
            pl.BlockSpec((d_in, dout), lambda i: (0, 0)),
            pl.BlockSpec((d_in, fr), lambda i: (0, 0)),
            pl.BlockSpec((fr,), lambda i: (0,)),
            pl.BlockSpec((_NC, r), lambda i: (0, i)),
        ],
        out_specs=[
            pl.BlockSpec((r, dout), lambda i: (i, 0)),
            pl.BlockSpec((r, 2 * rank), lambda i: (i, 0)),
            pl.BlockSpec((rank, rank), lambda i: (0, 0)),
            pl.BlockSpec((2, rank), lambda i: (0, 0)),
        ],
        out_shape=[
            jax.ShapeDtypeStruct((npad, dout), _F32),
            jax.ShapeDtypeStruct((npad, 2 * rank), _F32),
            jax.ShapeDtypeStruct((rank, rank), _F32),
            jax.ShapeDtypeStruct((2, rank), _F32),
        ],
    )(x, W_gcn, W_att, b_att, deg_parts)

    s0, s1 = _agg_kernel(npad, dout, nbt, rpt)(er, h2)

    out = pl.pallas_call(
        _tcc_body(n, dout, rank),
        grid=(g,),
        in_specs=[
            pl.BlockSpec((r, dout), lambda i: (i, 0)),
            pl.BlockSpec((r, dout), lambda i: (i, 0)),
            pl.BlockSpec((r, dout), lambda i: (i, 0)),
            pl.BlockSpec((r, 2 * rank), lambda i: (i, 0)),
            pl.BlockSpec((_NC, r), lambda i: (0, i)),
            pl.BlockSpec((rank, rank), lambda i: (0, 0)),
            pl.BlockSpec((2, rank), lambda i: (0, 0)),
            pl.BlockSpec((dout + 2 * rank, dout), lambda i: (0, 0)),
            pl.BlockSpec((dout,), lambda i: (0,)),
            pl.BlockSpec((dout,), lambda i: (0,)),
        ],
        out_specs=pl.BlockSpec((r, dout), lambda i: (i, 0)),
        out_shape=jax.ShapeDtypeStruct((n, dout), _F32),
    )(s0, s1, h2, tmp, deg_parts, vtz, cucv, W_red, b_gcn, b_red)

    return out
